# Initial kernel scaffold; baseline (speedup 1.0000x reference)
import jax
import jax.numpy as jnp
from jax.experimental import pallas as pl


def kernel(input_ids, vocab_ids, length, llm_table, code_table):
    raise NotImplementedError("write your pallas kernel here")


# speedup vs baseline: 1.9671x; 1.9671x over previous
import functools

import jax
import jax.numpy as jnp
from jax import lax
from jax.experimental import pallas as pl
from jax.experimental.pallas import tpu as pltpu
from jax.experimental.pallas import tpu_sc as plsc

VOCAB = 100000
D = 1024
B, L = 4, 2048
N = B * L
NW = 32
CHUNK = N // NW
S = 16
NSUB = CHUNK // S
GRP = D // 16

_mesh = plsc.VectorSubcoreMesh(core_axis_name="c", subcore_axis_name="s")


@functools.partial(
    pl.kernel,
    out_type=[
        jax.ShapeDtypeStruct((N,), jnp.int32),
        jax.ShapeDtypeStruct((N,), jnp.int32),
        jax.ShapeDtypeStruct((N,), jnp.int32),
        jax.ShapeDtypeStruct((N,), jnp.int32),
        jax.ShapeDtypeStruct((N,), jnp.int32),
        jax.ShapeDtypeStruct((N, D), jnp.float32),
    ],
    mesh=_mesh,
    scratch_types=[
        pltpu.VMEM((CHUNK,), jnp.int32),
        pltpu.VMEM((CHUNK,), jnp.int32),
        pltpu.VMEM((16,), jnp.int32),
        pltpu.VMEM((CHUNK,), jnp.int32),
        pltpu.VMEM((CHUNK,), jnp.int32),
        pltpu.VMEM((CHUNK,), jnp.int32),
        pltpu.VMEM((CHUNK,), jnp.int32),
        pltpu.VMEM((CHUNK,), jnp.int32),
        pltpu.VMEM((S, D), jnp.float32),
        pltpu.VMEM((S, D), jnp.float32),
        pltpu.SemaphoreType.DMA,
        pltpu.SemaphoreType.DMA,
    ],
)
def _emb_kernel(ids_h, vids_h, len_h, llm_h, code_h,
                mll_h, mcd_h, lin_h, cin_h, att_h, emb_h,
                ids_v, vids_v, len_v, mll_v, mcd_v, lin_v, cin_v, att_v,
                rows_a, rows_b, sem_a, sem_b):
    c = lax.axis_index("c")
    s = lax.axis_index("s")
    wid = s * 2 + c
    base = wid * CHUNK

    pltpu.sync_copy(ids_h.at[pl.ds(base, CHUNK)], ids_v)
    pltpu.sync_copy(vids_h.at[pl.ds(base, CHUNK)], vids_v)
    pltpu.sync_copy(len_h.at[wid], len_v)

    iot = lax.iota(jnp.int32, 16)
    lenb = len_v[...]
    pos_base = base % L

    def jbody(j, carry):
        sl = pl.ds(j * S, S)
        ids = ids_v[sl]
        vid = vids_v[sl]
        pos = pos_base + j * S + iot
        one = jnp.ones((16,), jnp.int32)
        zero = jnp.zeros((16,), jnp.int32)
        attn = jnp.where(pos < lenb, one, zero)
        m_cod = attn * vid
        m_llm = attn - m_cod
        llm_idx = ids * m_llm
        cod_idx = ids * m_cod
        mll_v[sl] = m_llm
        mcd_v[sl] = m_cod
        att_v[sl] = attn
        lin_v[sl] = llm_idx
        cin_v[sl] = cod_idx
        cp_a = pltpu.async_copy(llm_h.at[llm_idx], rows_a, sem_a)
        cp_b = pltpu.async_copy(code_h.at[cod_idx], rows_b, sem_b)
        cp_a.wait()
        cp_b.wait()

        def addbody(i, carry2):
            r = i >> 3
            cb = (i & 7) * 128
            for k in range(8):
                csl = pl.ds(cb + k * 16, 16)
                rows_a[r, csl] = rows_a[r, csl] + rows_b[r, csl]
            return carry2

        lax.fori_loop(0, S * 8, addbody, 0)
        pltpu.sync_copy(rows_a, emb_h.at[pl.ds(base + j * S, S)])
        return carry

    lax.fori_loop(0, NSUB, jbody, 0)

    pltpu.sync_copy(mll_v, mll_h.at[pl.ds(base, CHUNK)])
    pltpu.sync_copy(mcd_v, mcd_h.at[pl.ds(base, CHUNK)])
    pltpu.sync_copy(lin_v, lin_h.at[pl.ds(base, CHUNK)])
    pltpu.sync_copy(cin_v, cin_h.at[pl.ds(base, CHUNK)])
    pltpu.sync_copy(att_v, att_h.at[pl.ds(base, CHUNK)])


def kernel(input_ids, vocab_ids, length, llm_table, code_table):
    ids = input_ids.reshape(-1)
    vids = vocab_ids.reshape(-1)
    len_bcast = jnp.broadcast_to(
        jnp.repeat(length, NW // B)[:, None], (NW, 16))
    mll, mcd, lin, cin, att, emb = _emb_kernel(
        ids, vids, len_bcast, llm_table, code_table)
    shp = (B, L)
    return (mll.reshape(shp).astype(bool),
            mcd.reshape(shp).astype(bool),
            lin.reshape(shp),
            cin.reshape(shp),
            att.reshape(shp).astype(bool),
            emb.reshape(B, L, D))
